# baseline (speedup 1.0000x reference)
import functools

import jax
import jax.numpy as jnp
from jax.experimental import pallas as pl

F32 = jnp.float32


def _lrelu(x):
    return jnp.where(x >= 0, x, 0.2 * x)


def _conv_body(x_ref, w_ref, b_ref, o_ref, *, THo, Wo2, Ci2, Co):
    t = pl.program_id(1)
    r0 = t * THo
    y = None
    for q in (0, 1):
        for s in (0, 1):
            acc = jnp.zeros((THo * Wo2, Co), F32)
            for dy in range(3):
                for p in (0, 1):
                    xs = x_ref[0, pl.ds(r0 + (q + dy) // 2, THo),
                               (q + dy) % 2, p:p + Wo2, :]
                    xs = xs.reshape(THo * Wo2, Ci2)
                    acc = acc + jnp.dot(xs, w_ref[(s * 3 + dy) * 2 + p],
                                        preferred_element_type=F32)
            y = acc if y is None else jnp.maximum(y, acc)
    y = _lrelu(y + b_ref[0][None, :])
    o_ref[0] = y.reshape(THo, Wo2, Co)


def _conv(x, w, b, *, THo=14):
    B, H, W, Ci = x.shape
    Co = w.shape[3]
    xp = jnp.pad(x, ((0, 0), (1, 1), (1, 1), (0, 0)))
    xf = xp.reshape(B, (H + 2) // 2, 2, (W + 2) // 2, 2 * Ci)
    z = jnp.zeros((Ci, Co), F32)
    mats = []
    for s in (0, 1):
        for dy in range(3):
            for p in (0, 1):
                half = [z, z]
                for dx in range(3):
                    if (s + dx) // 2 == p:
                        half[(s + dx) % 2] = w[dy, dx]
                mats.append(jnp.concatenate(half, axis=0))
    wr = jnp.stack(mats, axis=0)
    br = b.reshape(1, Co)
    Ho, Wo = H // 2, W // 2
    body = functools.partial(_conv_body, THo=THo, Wo2=Wo, Ci2=2 * Ci, Co=Co)
    return pl.pallas_call(
        body,
        grid=(B, Ho // THo),
        in_specs=[
            pl.BlockSpec((1,) + xf.shape[1:],
                         lambda bb, tt: (bb, 0, 0, 0, 0)),
            pl.BlockSpec(wr.shape, lambda bb, tt: (0, 0, 0)),
            pl.BlockSpec(br.shape, lambda bb, tt: (0, 0)),
        ],
        out_specs=pl.BlockSpec((1, THo, Wo, Co),
                               lambda bb, tt: (bb, tt, 0, 0)),
        out_shape=jax.ShapeDtypeStruct((B, Ho, Wo, Co), F32),
    )(xf, wr, br)


def _enc1_body(x_ref, w_ref, b_ref, o_ref, *, TH, Wout, Co):
    t = pl.program_id(1)
    h0 = t * TH
    bias = b_ref[0][None, :]
    y = None
    for pi, py in enumerate((0, 1)):
        for pj, px in enumerate((0, 1)):
            acc = jnp.zeros((TH * Wout, Co), F32)
            for a in range(2):
                for c in range(2):
                    xs = x_ref[0, pl.ds(h0 + a + py, TH),
                               c + px:c + px + Wout, :]
                    xs = xs.reshape(TH * Wout, 12)
                    wi = w_ref[((pi * 2 + pj) * 2 + a) * 2 + c]
                    acc = acc + jnp.dot(xs, wi, preferred_element_type=F32)
            y = acc if y is None else jnp.maximum(y, acc)
    y = _lrelu(y + bias)
    o_ref[0] = y.reshape(TH, Wout, Co)


def _grouped3x3_mats(w):
    Ci, Co = w.shape[2], w.shape[3]
    zb = jnp.zeros((Ci, Co), F32)
    mats = []
    for py in (0, 1):
        for px in (0, 1):
            rmap = {}
            for e in (-1, 0, 1):
                t_g = (py + e) // 2
                a = t_g + 1 if py == 0 else t_g
                rmap[(a, (py + e) % 2)] = e + 1
            cmap = {}
            for e in (-1, 0, 1):
                t_g = (px + e) // 2
                c = t_g + 1 if px == 0 else t_g
                cmap[(c, (px + e) % 2)] = e + 1
            for a in range(2):
                for c in range(2):
                    blocks = []
                    for pr in range(2):
                        for pc in range(2):
                            dy = rmap.get((a, pr))
                            dx = cmap.get((c, pc))
                            if dy is None or dx is None:
                                blocks.append(zb)
                            else:
                                blocks.append(w[dy, dx])
                    mats.append(jnp.concatenate(blocks, axis=0))
    return jnp.stack(mats, axis=0)


def _enc1(x, w, b, *, TH=28):
    B = x.shape[0]
    Co = w.shape[3]
    xg = x.reshape(B, 112, 2, 112, 2, 3)
    xg = jnp.transpose(xg, (0, 1, 3, 2, 4, 5)).reshape(B, 112, 112, 12)
    xg = jnp.pad(xg, ((0, 0), (1, 1), (1, 1), (0, 0)))
    wg = _grouped3x3_mats(w)
    body = functools.partial(_enc1_body, TH=TH, Wout=112, Co=Co)
    return pl.pallas_call(
        body,
        grid=(B, 112 // TH),
        in_specs=[
            pl.BlockSpec((1, 114, 114, 12), lambda bb, tt: (bb, 0, 0, 0)),
            pl.BlockSpec(wg.shape, lambda bb, tt: (0, 0, 0)),
            pl.BlockSpec((1, Co), lambda bb, tt: (0, 0)),
        ],
        out_specs=pl.BlockSpec((1, TH, 112, Co), lambda bb, tt: (bb, tt, 0, 0)),
        out_shape=jax.ShapeDtypeStruct((B, 112, 112, Co), F32),
    )(xg, wg, b.reshape(1, Co))


def _deconv_body(x_ref, wf_ref, b_ref, o_ref, *, TH, W, Ci, Co, grouped_out):
    t = pl.program_id(1)
    i0 = t * TH
    xs = x_ref[0, pl.ds(i0, TH + 2), :, :]
    xs = xs.reshape((TH + 2) * (W + 2), Ci)
    u = jnp.dot(xs, wf_ref[...], preferred_element_type=F32)
    u = u.reshape(TH + 2, W + 2, 16 * Co)
    bias = b_ref[0]

    def up(r, s):
        k = (4 * r + s) * Co
        return u[:, :, k:k + Co]

    ph = {}
    for py in range(2):
        for px in range(2):
            v = bias[None, None, :]
            for a in range(2):
                for bb in range(2):
                    v = v + up(py + 2 * a, px + 2 * bb)[
                        py + a:py + a + TH, px + bb:px + bb + W]
            ph[(py, px)] = _lrelu(v)
    if grouped_out:
        y = jnp.concatenate(
            [ph[(0, 0)], ph[(0, 1)], ph[(1, 0)], ph[(1, 1)]], axis=-1)
        o_ref[0] = y
    else:
        r0 = jnp.concatenate(
            [ph[(0, 0)][:, :, None, :], ph[(0, 1)][:, :, None, :]], axis=2)
        r1 = jnp.concatenate(
            [ph[(1, 0)][:, :, None, :], ph[(1, 1)][:, :, None, :]], axis=2)
        y = jnp.concatenate([r0[:, None], r1[:, None]], axis=1)
        o_ref[0] = y.reshape(2 * TH, 2 * W, Co)


def _deconv(x, w, b, *, TH=14, grouped_out=False):
    B, H, W, Ci = x.shape
    Co = w.shape[3]
    xp = jnp.pad(x, ((0, 0), (1, 1), (1, 1), (0, 0)))
    wf = jnp.transpose(w, (2, 0, 1, 3)).reshape(Ci, 16 * Co)
    br = b.reshape(1, Co)
    body = functools.partial(_deconv_body, TH=TH, W=W, Ci=Ci, Co=Co,
                             grouped_out=grouped_out)
    if grouped_out:
        ospec = pl.BlockSpec((1, TH, W, 4 * Co), lambda bb, tt: (bb, tt, 0, 0))
        oshape = jax.ShapeDtypeStruct((B, H, W, 4 * Co), F32)
    else:
        ospec = pl.BlockSpec((1, 2 * TH, 2 * W, Co),
                             lambda bb, tt: (bb, tt, 0, 0))
        oshape = jax.ShapeDtypeStruct((B, 2 * H, 2 * W, Co), F32)
    return pl.pallas_call(
        body,
        grid=(B, H // TH),
        in_specs=[
            pl.BlockSpec((1, H + 2, W + 2, Ci), lambda bb, tt: (bb, 0, 0, 0)),
            pl.BlockSpec(wf.shape, lambda bb, tt: (0, 0)),
            pl.BlockSpec(br.shape, lambda bb, tt: (0, 0)),
        ],
        out_specs=ospec,
        out_shape=oshape,
    )(xp, wf, br)


def _vq_body(x_ref, w_ref, b_ref, cbt_ref, cb_ref, o_ref):
    zp = jax.nn.sigmoid(
        jnp.dot(x_ref[...], w_ref[...], preferred_element_type=F32)
        + b_ref[0][None, :])
    cbt = cbt_ref[...]
    cbsq = jnp.sum(cbt * cbt, axis=0, keepdims=True)
    d = cbsq - 2.0 * jnp.dot(zp, cbt, preferred_element_type=F32)
    dmin = jnp.min(d, axis=1, keepdims=True)
    iota = jax.lax.broadcasted_iota(jnp.int32, d.shape, 1)
    big = jnp.int32(d.shape[1])
    masked = jnp.where(d <= dmin, iota, big)
    idx = jnp.min(masked, axis=1, keepdims=True)
    oh = (iota == idx).astype(F32)
    o_ref[...] = jnp.dot(oh, cb_ref[...], preferred_element_type=F32)


def _vq(z, w5, b5, codebook):
    M = z.shape[0]
    C = codebook.shape[1]
    return pl.pallas_call(
        _vq_body,
        out_shape=jax.ShapeDtypeStruct((M, C), F32),
    )(z, w5, b5.reshape(1, C), codebook.T, codebook)


def _final_body(x_ref, w9_ref, b1_ref, w2_ref, b2_ref, o_ref, *, TH, Wout):
    acc = jnp.zeros((TH * Wout, 128), F32)
    for rs in range(3):
        for cs in range(3):
            xs = x_ref[0, rs:rs + TH, cs:cs + Wout, :]
            xs = xs.reshape(TH * Wout, 256)
            acc = acc + jnp.dot(xs, w9_ref[rs * 3 + cs],
                                preferred_element_type=F32)
    y = _lrelu(acc + b1_ref[0][None, :])
    z = jax.nn.sigmoid(
        jnp.dot(y, w2_ref[...], preferred_element_type=F32)
        + b2_ref[0][None, :])
    o_ref[0] = z.reshape(TH, Wout, 12)


def _final(xg, w1, b1, w2, b2, *, TH=28):
    B = xg.shape[0]
    Wg = xg.shape[2]
    xp = jnp.pad(xg, ((0, 0), (1, 1), (1, 1), (0, 0)))
    nt = 112 // TH
    xt = jnp.stack([xp[:, i * TH:i * TH + TH + 2] for i in range(nt)], axis=1)
    xt = xt.reshape(B * nt, TH + 2, Wg + 2, 256)
    mats = _grouped3x3_mats(w1)
    z32 = jnp.zeros((256, 32), F32)
    w9 = []
    for rs in range(3):
        for cs in range(3):
            cols = []
            for py in (0, 1):
                for px in (0, 1):
                    a, c = rs - py, cs - px
                    if 0 <= a <= 1 and 0 <= c <= 1:
                        cols.append(mats[((py * 2 + px) * 2 + a) * 2 + c])
                    else:
                        cols.append(z32)
            w9.append(jnp.concatenate(cols, axis=1))
    w9 = jnp.stack(w9, axis=0)
    b1t = jnp.tile(b1, 4).reshape(1, 128)
    w2bd = jax.scipy.linalg.block_diag(*([w2.reshape(32, 3)] * 4))
    b2t = jnp.tile(b2, 4).reshape(1, 12)
    body = functools.partial(_final_body, TH=TH, Wout=Wg)
    y = pl.pallas_call(
        body,
        grid=(B * nt,),
        in_specs=[
            pl.BlockSpec((1, TH + 2, Wg + 2, 256), lambda g: (g, 0, 0, 0)),
            pl.BlockSpec(w9.shape, lambda g: (0, 0, 0)),
            pl.BlockSpec((1, 128), lambda g: (0, 0)),
            pl.BlockSpec(w2bd.shape, lambda g: (0, 0)),
            pl.BlockSpec((1, 12), lambda g: (0, 0)),
        ],
        out_specs=pl.BlockSpec((1, TH, Wg, 12), lambda g: (g, 0, 0, 0)),
        out_shape=jax.ShapeDtypeStruct((B * nt, TH, Wg, 12), F32),
    )(xt, w9, b1t, w2bd, b2t)
    return y.reshape(B, 112, 112, 12)


def kernel(input, enc_params, dec_deconv, dec_conv, codebook):
    x = jnp.transpose(input, (0, 2, 3, 1))
    B = x.shape[0]
    h = _conv(x, enc_params[0][0], enc_params[0][1])
    for i in (1, 2, 3):
        h = _conv(h, enc_params[i][0], enc_params[i][1])
    w5, b5 = enc_params[4]
    M = B * h.shape[1] * h.shape[2]
    q = _vq(h.reshape(M, h.shape[3]), w5.reshape(w5.shape[2], w5.shape[3]),
            b5, codebook)
    qz = q.reshape(B, h.shape[1], h.shape[2], codebook.shape[1])
    qz = _deconv(qz, dec_deconv[0][0], dec_deconv[0][1], TH=14)
    qz = _deconv(qz, dec_deconv[1][0], dec_deconv[1][1], TH=14)
    qz = _deconv(qz, dec_deconv[2][0], dec_deconv[2][1], TH=14)
    qz = _deconv(qz, dec_deconv[3][0], dec_deconv[3][1], TH=14,
                 grouped_out=True)
    y = _final(qz, dec_conv[0][0], dec_conv[0][1], dec_conv[1][0],
               dec_conv[1][1])
    y = y.reshape(B, 112, 112, 2, 2, 3)
    return jnp.transpose(y, (0, 5, 1, 3, 2, 4)).reshape(B, 3, 224, 224)


# speedup vs baseline: 1.1131x; 1.1131x over previous
import functools

import jax
import jax.numpy as jnp
from jax.experimental import pallas as pl

F32 = jnp.float32


def _lrelu(x):
    return jnp.where(x >= 0, x, 0.2 * x)


def _conv_tap_body(x_ref, w_ref, b_ref, o_ref, *, taps, TH, Wout, Ci, Co):
    t = pl.program_id(1)
    h0 = t * TH
    acc = jnp.zeros((TH * Wout, Co), F32)
    for i, (dy, dx) in enumerate(taps):
        xs = x_ref[0, pl.ds(h0 + dy, TH), dx:dx + Wout, :]
        xs = xs.reshape(TH * Wout, Ci)
        acc = acc + jnp.dot(xs, w_ref[i], preferred_element_type=F32)
    y = acc + b_ref[0][None, :]
    y = y.reshape(TH, Wout, Co)
    y = y.reshape(TH // 2, 2, Wout, Co)
    y = jnp.max(y, axis=1)
    y = y.reshape(TH // 2, Wout // 2, 2, Co)
    y = jnp.max(y, axis=2)
    y = _lrelu(y)
    o_ref[0] = y


def _conv_tap(x, w, b, *, TH=28):
    B, H, W, Ci = x.shape
    kh, kw, _, Co = w.shape
    xp = jnp.pad(x, ((0, 0), (1, 1), (1, 1), (0, 0)))
    taps = [(dy, dx) for dy in range(kh) for dx in range(kw)]
    wr = w.reshape(kh * kw, Ci, Co)
    br = b.reshape(1, Co)
    body = functools.partial(_conv_tap_body, taps=taps, TH=TH, Wout=W,
                             Ci=Ci, Co=Co)
    return pl.pallas_call(
        body,
        grid=(B, H // TH),
        in_specs=[
            pl.BlockSpec((1, H + 2, W + 2, Ci), lambda bb, tt: (bb, 0, 0, 0)),
            pl.BlockSpec(wr.shape, lambda bb, tt: (0, 0, 0)),
            pl.BlockSpec(br.shape, lambda bb, tt: (0, 0)),
        ],
        out_specs=pl.BlockSpec((1, TH // 2, W // 2, Co),
                               lambda bb, tt: (bb, tt, 0, 0)),
        out_shape=jax.ShapeDtypeStruct((B, H // 2, W // 2, Co), F32),
    )(xp, wr, br)


def _conv_body(x_ref, w_ref, b_ref, o_ref, *, THo, Wo2, Ci2, Co):
    t = pl.program_id(1)
    r0 = t * THo
    y = None
    for q in (0, 1):
        for s in (0, 1):
            acc = jnp.zeros((THo * Wo2, Co), F32)
            for dy in range(3):
                for p in (0, 1):
                    xs = x_ref[0, pl.ds(r0 + (q + dy) // 2, THo),
                               (q + dy) % 2, p:p + Wo2, :]
                    xs = xs.reshape(THo * Wo2, Ci2)
                    acc = acc + jnp.dot(xs, w_ref[(s * 3 + dy) * 2 + p],
                                        preferred_element_type=F32)
            y = acc if y is None else jnp.maximum(y, acc)
    y = _lrelu(y + b_ref[0][None, :])
    o_ref[0] = y.reshape(THo, Wo2, Co)


def _conv(x, w, b, *, THo=14):
    B, H, W, Ci = x.shape
    Co = w.shape[3]
    xp = jnp.pad(x, ((0, 0), (1, 1), (1, 1), (0, 0)))
    xf = xp.reshape(B, (H + 2) // 2, 2, (W + 2) // 2, 2 * Ci)
    z = jnp.zeros((Ci, Co), F32)
    mats = []
    for s in (0, 1):
        for dy in range(3):
            for p in (0, 1):
                half = [z, z]
                for dx in range(3):
                    if (s + dx) // 2 == p:
                        half[(s + dx) % 2] = w[dy, dx]
                mats.append(jnp.concatenate(half, axis=0))
    wr = jnp.stack(mats, axis=0)
    br = b.reshape(1, Co)
    Ho, Wo = H // 2, W // 2
    body = functools.partial(_conv_body, THo=THo, Wo2=Wo, Ci2=2 * Ci, Co=Co)
    return pl.pallas_call(
        body,
        grid=(B, Ho // THo),
        in_specs=[
            pl.BlockSpec((1,) + xf.shape[1:],
                         lambda bb, tt: (bb, 0, 0, 0, 0)),
            pl.BlockSpec(wr.shape, lambda bb, tt: (0, 0, 0)),
            pl.BlockSpec(br.shape, lambda bb, tt: (0, 0)),
        ],
        out_specs=pl.BlockSpec((1, THo, Wo, Co),
                               lambda bb, tt: (bb, tt, 0, 0)),
        out_shape=jax.ShapeDtypeStruct((B, Ho, Wo, Co), F32),
    )(xf, wr, br)


def _enc1_body(x_ref, w_ref, b_ref, o_ref, *, TH, Wout, Co):
    t = pl.program_id(1)
    h0 = t * TH
    bias = b_ref[0][None, :]
    y = None
    for pi, py in enumerate((0, 1)):
        for pj, px in enumerate((0, 1)):
            acc = jnp.zeros((TH * Wout, Co), F32)
            for a in range(2):
                for c in range(2):
                    xs = x_ref[0, pl.ds(h0 + a + py, TH),
                               c + px:c + px + Wout, :]
                    xs = xs.reshape(TH * Wout, 12)
                    wi = w_ref[((pi * 2 + pj) * 2 + a) * 2 + c]
                    acc = acc + jnp.dot(xs, wi, preferred_element_type=F32)
            y = acc if y is None else jnp.maximum(y, acc)
    y = _lrelu(y + bias)
    o_ref[0] = y.reshape(TH, Wout, Co)


def _grouped3x3_mats(w):
    Ci, Co = w.shape[2], w.shape[3]
    zb = jnp.zeros((Ci, Co), F32)
    mats = []
    for py in (0, 1):
        for px in (0, 1):
            rmap = {}
            for e in (-1, 0, 1):
                t_g = (py + e) // 2
                a = t_g + 1 if py == 0 else t_g
                rmap[(a, (py + e) % 2)] = e + 1
            cmap = {}
            for e in (-1, 0, 1):
                t_g = (px + e) // 2
                c = t_g + 1 if px == 0 else t_g
                cmap[(c, (px + e) % 2)] = e + 1
            for a in range(2):
                for c in range(2):
                    blocks = []
                    for pr in range(2):
                        for pc in range(2):
                            dy = rmap.get((a, pr))
                            dx = cmap.get((c, pc))
                            if dy is None or dx is None:
                                blocks.append(zb)
                            else:
                                blocks.append(w[dy, dx])
                    mats.append(jnp.concatenate(blocks, axis=0))
    return jnp.stack(mats, axis=0)


def _enc1(x, w, b, *, TH=28):
    B = x.shape[0]
    Co = w.shape[3]
    xg = x.reshape(B, 112, 2, 112, 2, 3)
    xg = jnp.transpose(xg, (0, 1, 3, 2, 4, 5)).reshape(B, 112, 112, 12)
    xg = jnp.pad(xg, ((0, 0), (1, 1), (1, 1), (0, 0)))
    wg = _grouped3x3_mats(w)
    body = functools.partial(_enc1_body, TH=TH, Wout=112, Co=Co)
    return pl.pallas_call(
        body,
        grid=(B, 112 // TH),
        in_specs=[
            pl.BlockSpec((1, 114, 114, 12), lambda bb, tt: (bb, 0, 0, 0)),
            pl.BlockSpec(wg.shape, lambda bb, tt: (0, 0, 0)),
            pl.BlockSpec((1, Co), lambda bb, tt: (0, 0)),
        ],
        out_specs=pl.BlockSpec((1, TH, 112, Co), lambda bb, tt: (bb, tt, 0, 0)),
        out_shape=jax.ShapeDtypeStruct((B, 112, 112, Co), F32),
    )(xg, wg, b.reshape(1, Co))


def _deconv_body(x_ref, wf_ref, b_ref, o_ref, *, TH, W, Ci, Co, grouped_out):
    t = pl.program_id(1)
    i0 = t * TH
    xs = x_ref[0, pl.ds(i0, TH + 2), :, :]
    xs = xs.reshape((TH + 2) * (W + 2), Ci)
    u = jnp.dot(xs, wf_ref[...], preferred_element_type=F32)
    u = u.reshape(TH + 2, W + 2, 16 * Co)
    bias = b_ref[0]

    def up(r, s):
        k = (4 * r + s) * Co
        return u[:, :, k:k + Co]

    ph = {}
    for py in range(2):
        for px in range(2):
            v = bias[None, None, :]
            for a in range(2):
                for bb in range(2):
                    v = v + up(py + 2 * a, px + 2 * bb)[
                        py + a:py + a + TH, px + bb:px + bb + W]
            ph[(py, px)] = _lrelu(v)
    if grouped_out:
        y = jnp.concatenate(
            [ph[(0, 0)], ph[(0, 1)], ph[(1, 0)], ph[(1, 1)]], axis=-1)
        o_ref[0] = y
    else:
        r0 = jnp.concatenate(
            [ph[(0, 0)][:, :, None, :], ph[(0, 1)][:, :, None, :]], axis=2)
        r1 = jnp.concatenate(
            [ph[(1, 0)][:, :, None, :], ph[(1, 1)][:, :, None, :]], axis=2)
        y = jnp.concatenate([r0[:, None], r1[:, None]], axis=1)
        o_ref[0] = y.reshape(2 * TH, 2 * W, Co)


def _deconv(x, w, b, *, TH=14, grouped_out=False):
    B, H, W, Ci = x.shape
    Co = w.shape[3]
    xp = jnp.pad(x, ((0, 0), (1, 1), (1, 1), (0, 0)))
    wf = jnp.transpose(w, (2, 0, 1, 3)).reshape(Ci, 16 * Co)
    br = b.reshape(1, Co)
    body = functools.partial(_deconv_body, TH=TH, W=W, Ci=Ci, Co=Co,
                             grouped_out=grouped_out)
    if grouped_out:
        ospec = pl.BlockSpec((1, TH, W, 4 * Co), lambda bb, tt: (bb, tt, 0, 0))
        oshape = jax.ShapeDtypeStruct((B, H, W, 4 * Co), F32)
    else:
        ospec = pl.BlockSpec((1, 2 * TH, 2 * W, Co),
                             lambda bb, tt: (bb, tt, 0, 0))
        oshape = jax.ShapeDtypeStruct((B, 2 * H, 2 * W, Co), F32)
    return pl.pallas_call(
        body,
        grid=(B, H // TH),
        in_specs=[
            pl.BlockSpec((1, H + 2, W + 2, Ci), lambda bb, tt: (bb, 0, 0, 0)),
            pl.BlockSpec(wf.shape, lambda bb, tt: (0, 0)),
            pl.BlockSpec(br.shape, lambda bb, tt: (0, 0)),
        ],
        out_specs=ospec,
        out_shape=oshape,
    )(xp, wf, br)


def _vq_body(x_ref, w_ref, b_ref, cbt_ref, cb_ref, o_ref):
    zp = jax.nn.sigmoid(
        jnp.dot(x_ref[...], w_ref[...], preferred_element_type=F32)
        + b_ref[0][None, :])
    cbt = cbt_ref[...]
    cbsq = jnp.sum(cbt * cbt, axis=0, keepdims=True)
    d = cbsq - 2.0 * jnp.dot(zp, cbt, preferred_element_type=F32)
    dmin = jnp.min(d, axis=1, keepdims=True)
    iota = jax.lax.broadcasted_iota(jnp.int32, d.shape, 1)
    big = jnp.int32(d.shape[1])
    masked = jnp.where(d <= dmin, iota, big)
    idx = jnp.min(masked, axis=1, keepdims=True)
    oh = (iota == idx).astype(F32)
    o_ref[...] = jnp.dot(oh, cb_ref[...], preferred_element_type=F32)


def _vq(z, w5, b5, codebook):
    M = z.shape[0]
    C = codebook.shape[1]
    return pl.pallas_call(
        _vq_body,
        out_shape=jax.ShapeDtypeStruct((M, C), F32),
    )(z, w5, b5.reshape(1, C), codebook.T, codebook)


def _final_body(x_ref, w9_ref, b1_ref, w2_ref, b2_ref, o_ref, *, TH, Wout):
    acc = jnp.zeros((TH * Wout, 128), F32)
    for rs in range(3):
        for cs in range(3):
            xs = x_ref[0, rs:rs + TH, cs:cs + Wout, :]
            xs = xs.reshape(TH * Wout, 256)
            acc = acc + jnp.dot(xs, w9_ref[rs * 3 + cs],
                                preferred_element_type=F32)
    y = _lrelu(acc + b1_ref[0][None, :])
    z = jax.nn.sigmoid(
        jnp.dot(y, w2_ref[...], preferred_element_type=F32)
        + b2_ref[0][None, :])
    o_ref[0] = z.reshape(TH, Wout, 12)


def _final(xg, w1, b1, w2, b2, *, TH=28):
    B = xg.shape[0]
    Wg = xg.shape[2]
    xp = jnp.pad(xg, ((0, 0), (1, 1), (1, 1), (0, 0)))
    nt = 112 // TH
    xt = jnp.stack([xp[:, i * TH:i * TH + TH + 2] for i in range(nt)], axis=1)
    xt = xt.reshape(B * nt, TH + 2, Wg + 2, 256)
    mats = _grouped3x3_mats(w1)
    z32 = jnp.zeros((256, 32), F32)
    w9 = []
    for rs in range(3):
        for cs in range(3):
            cols = []
            for py in (0, 1):
                for px in (0, 1):
                    a, c = rs - py, cs - px
                    if 0 <= a <= 1 and 0 <= c <= 1:
                        cols.append(mats[((py * 2 + px) * 2 + a) * 2 + c])
                    else:
                        cols.append(z32)
            w9.append(jnp.concatenate(cols, axis=1))
    w9 = jnp.stack(w9, axis=0)
    b1t = jnp.tile(b1, 4).reshape(1, 128)
    w2bd = jax.scipy.linalg.block_diag(*([w2.reshape(32, 3)] * 4))
    b2t = jnp.tile(b2, 4).reshape(1, 12)
    body = functools.partial(_final_body, TH=TH, Wout=Wg)
    y = pl.pallas_call(
        body,
        grid=(B * nt,),
        in_specs=[
            pl.BlockSpec((1, TH + 2, Wg + 2, 256), lambda g: (g, 0, 0, 0)),
            pl.BlockSpec(w9.shape, lambda g: (0, 0, 0)),
            pl.BlockSpec((1, 128), lambda g: (0, 0)),
            pl.BlockSpec(w2bd.shape, lambda g: (0, 0)),
            pl.BlockSpec((1, 12), lambda g: (0, 0)),
        ],
        out_specs=pl.BlockSpec((1, TH, Wg, 12), lambda g: (g, 0, 0, 0)),
        out_shape=jax.ShapeDtypeStruct((B * nt, TH, Wg, 12), F32),
    )(xt, w9, b1t, w2bd, b2t)
    return y.reshape(B, 112, 112, 12)


def kernel(input, enc_params, dec_deconv, dec_conv, codebook):
    x = jnp.transpose(input, (0, 2, 3, 1))
    B = x.shape[0]
    h = _enc1(x, enc_params[0][0], enc_params[0][1])
    for i in (1, 2, 3):
        h = _conv_tap(h, enc_params[i][0], enc_params[i][1])
    w5, b5 = enc_params[4]
    M = B * h.shape[1] * h.shape[2]
    q = _vq(h.reshape(M, h.shape[3]), w5.reshape(w5.shape[2], w5.shape[3]),
            b5, codebook)
    qz = q.reshape(B, h.shape[1], h.shape[2], codebook.shape[1])
    qz = _deconv(qz, dec_deconv[0][0], dec_deconv[0][1], TH=14)
    qz = _deconv(qz, dec_deconv[1][0], dec_deconv[1][1], TH=14)
    qz = _deconv(qz, dec_deconv[2][0], dec_deconv[2][1], TH=14)
    qz = _deconv(qz, dec_deconv[3][0], dec_deconv[3][1], TH=14,
                 grouped_out=True)
    y = _final(qz, dec_conv[0][0], dec_conv[0][1], dec_conv[1][0],
               dec_conv[1][1])
    y = y.reshape(B, 112, 112, 2, 2, 3)
    return jnp.transpose(y, (0, 5, 1, 3, 2, 4)).reshape(B, 3, 224, 224)
